# baseline (speedup 1.0000x reference)
import functools

import jax
import jax.numpy as jnp
from jax import lax
from jax.experimental import pallas as pl
from jax.experimental.pallas import tpu as pltpu
from jax.experimental.pallas import tpu_sc as plsc

V, L, D, B = 1000, 50, 128, 1024
VP = 1024
NC, NS = 2, 16
NW = NC * NS
SEQ_PER_W = B // NW
CH = 25
NCH = SEQ_PER_W * (L // CH)
NFULL = V // 16
TAIL = V - 16


def _precompute_body(emb_ref, w_ref, b_ref, pe_ref, tab_ref, ptab_ref):
    dn = (((1,), (1,)), ((), ()))
    tab_ref[...] = lax.dot_general(
        emb_ref[...], w_ref[...], dn,
        precision=lax.Precision.HIGHEST, preferred_element_type=jnp.float32)
    ptab_ref[...] = lax.dot_general(
        pe_ref[...], w_ref[...], dn,
        precision=lax.Precision.HIGHEST, preferred_element_type=jnp.float32
    ) + b_ref[...]


def _precompute(emb, wp, bp, pe):
    return pl.pallas_call(
        _precompute_body,
        out_shape=[
            jax.ShapeDtypeStruct((V, VP), jnp.float32),
            jax.ShapeDtypeStruct((L, VP), jnp.float32),
        ],
    )(emb, wp, bp, pe)


@functools.cache
def _make_sc_lookup():
    def body(ids_hbm, tab_hbm, ptab_hbm, out_hbm, idx_v, ptab_v, rows_v,
             orow_v, sem):
        c = lax.axis_index("c")
        s = lax.axis_index("s")
        w = s * NC + c
        pltpu.sync_copy(ids_hbm.at[pl.ds(w * NCH, NCH)], idx_v)
        pltpu.sync_copy(ptab_hbm, ptab_v)

        @pl.loop(0, NCH)
        def _chunk(u):
            i = u // 2
            h = u % 2
            batch = w * SEQ_PER_W + i
            pltpu.async_copy(tab_hbm.at[idx_v.at[u]], rows_v, sem).wait()

            @pl.loop(0, CH)
            def _row(r):
                t = h * CH + r
                for k in range(NFULL):
                    x = rows_v[r, k >> 3, pl.ds((k & 7) * 16, 16)]
                    p = ptab_v[t, k >> 3, pl.ds((k & 7) * 16, 16)]
                    orow_v[pl.ds(k * 16, 16)] = x + p
                xt = rows_v[r, 7, pl.ds(88, 16)]
                pt = ptab_v[t, 7, pl.ds(88, 16)]
                orow_v[pl.ds(TAIL, 16)] = xt + pt
                pltpu.sync_copy(orow_v, out_hbm.at[batch, t])

    return pl.kernel(
        body,
        out_type=jax.ShapeDtypeStruct((B, L, V), jnp.float32),
        mesh=plsc.VectorSubcoreMesh(
            core_axis_name="c", subcore_axis_name="s",
            num_cores=NC, num_subcores=NS),
        scratch_types=[
            pltpu.VMEM((NCH, CH), jnp.int32),
            pltpu.VMEM((L, 8, 128), jnp.float32),
            pltpu.VMEM((CH, 8, 128), jnp.float32),
            pltpu.VMEM((V,), jnp.float32),
            pltpu.SemaphoreType.DMA,
        ],
    )


def kernel(input_ids, embedding, W, b, pe):
    wp = jnp.pad(W, ((0, VP - V), (0, 0)))
    bp = jnp.pad(b, (0, VP - V)).reshape(1, VP)
    tab, ptab = _precompute(embedding, wp, bp, pe)
    tab3 = tab.reshape(V, 8, 128)
    ptab3 = ptab.reshape(L, 8, 128)
    ids2 = input_ids.astype(jnp.int32).reshape(B * L // CH, CH)
    return _make_sc_lookup()(ids2, tab3, ptab3)


# speedup vs baseline: 1.3433x; 1.3433x over previous
import functools

import jax
import jax.numpy as jnp
from jax import lax
from jax.experimental import pallas as pl
from jax.experimental.pallas import tpu as pltpu
from jax.experimental.pallas import tpu_sc as plsc

V, L, D, B = 1000, 50, 128, 1024
VP = 1024
NC, NS = 2, 16
NW = NC * NS
SEQ_PER_W = B // NW
TOK_W = SEQ_PER_W * L
CH = 25
NCHUNK = TOK_W // CH
NFULL = V // 16
TAIL = V - 16


def _precompute_body(emb_ref, w_ref, b_ref, pe_ref, tab_ref, ptab_ref):
    dn = (((1,), (1,)), ((), ()))
    tab_ref[...] = lax.dot_general(
        emb_ref[...], w_ref[...], dn,
        precision=lax.Precision.HIGHEST, preferred_element_type=jnp.float32)
    ptab_ref[...] = lax.dot_general(
        pe_ref[...], w_ref[...], dn,
        precision=lax.Precision.HIGHEST, preferred_element_type=jnp.float32
    ) + b_ref[...]


def _precompute(emb, wp, bp, pe):
    return pl.pallas_call(
        _precompute_body,
        out_shape=[
            jax.ShapeDtypeStruct((V, VP), jnp.float32),
            jax.ShapeDtypeStruct((L, VP), jnp.float32),
        ],
    )(emb, wp, bp, pe)


@functools.cache
def _make_sc_lookup():
    def body(ids_hbm, tab_hbm, ptab_hbm, out_hbm, idx_v, ptab_v,
             rows0, rows1, out0, out1, sg0, sg1, so0, so1):
        c = lax.axis_index("c")
        s = lax.axis_index("s")
        w = s * NC + c
        pltpu.sync_copy(ids_hbm.at[pl.ds(w * NCHUNK, NCHUNK)], idx_v)
        pltpu.sync_copy(ptab_hbm, ptab_v)

        def gather_src(u):
            return tab_hbm.at[idx_v.at[u]]

        def do_chunk(u, rows, sg, rows_nxt, sg_nxt, out_v, so):
            @pl.when(u + 1 < NCHUNK)
            def _():
                pltpu.async_copy(gather_src(u + 1), rows_nxt, sg_nxt)

            pltpu.make_async_copy(gather_src(u), rows, sg).wait()
            i = u // 2
            h = u % 2
            batch = w * SEQ_PER_W + i

            def compute_row(r, orow, so, do_wait):
                @pl.when(do_wait)
                def _():
                    pltpu.make_async_copy(orow, out_hbm.at[0, 0], so).wait()

                for k in range(NFULL):
                    x = rows[r, k >> 3, pl.ds((k & 7) * 16, 16)]
                    p = ptab_v[h * CH + r, k >> 3, pl.ds((k & 7) * 16, 16)]
                    orow[pl.ds(k * 16, 16)] = x + p
                xt = rows[r, 7, pl.ds(88, 16)]
                pt = ptab_v[h * CH + r, 7, pl.ds(88, 16)]
                orow[pl.ds(TAIL, 16)] = xt + pt
                pltpu.async_copy(orow, out_hbm.at[batch, h * CH + r], so)

            @pl.loop(0, CH // 2)
            def _rowpair(j):
                not_first = (u + j) > 0
                compute_row(2 * j, out0, so0, not_first)
                compute_row(2 * j + 1, out1, so1, not_first)

            compute_row(CH - 1, out0, so0, jnp.bool_(True))

        pltpu.async_copy(gather_src(0), rows0, sg0)

        @pl.loop(0, NCHUNK // 2)
        def _pair(v):
            do_chunk(2 * v, rows0, sg0, rows1, sg1, out0, so0)
            do_chunk(2 * v + 1, rows1, sg1, rows0, sg0, out1, so1)

        pltpu.make_async_copy(out0, out_hbm.at[0, 0], so0).wait()
        pltpu.make_async_copy(out1, out_hbm.at[0, 0], so1).wait()

    return pl.kernel(
        body,
        out_type=jax.ShapeDtypeStruct((B, L, V), jnp.float32),
        mesh=plsc.VectorSubcoreMesh(
            core_axis_name="c", subcore_axis_name="s",
            num_cores=NC, num_subcores=NS),
        scratch_types=[
            pltpu.VMEM((NCHUNK, CH), jnp.int32),
            pltpu.VMEM((L, 8, 128), jnp.float32),
            pltpu.VMEM((CH, 8, 128), jnp.float32),
            pltpu.VMEM((CH, 8, 128), jnp.float32),
            pltpu.VMEM((V,), jnp.float32),
            pltpu.VMEM((V,), jnp.float32),
            pltpu.SemaphoreType.DMA,
            pltpu.SemaphoreType.DMA,
            pltpu.SemaphoreType.DMA,
            pltpu.SemaphoreType.DMA,
        ],
    )


def kernel(input_ids, embedding, W, b, pe):
    wp = jnp.pad(W, ((0, VP - V), (0, 0)))
    bp = jnp.pad(b, (0, VP - V)).reshape(1, VP)
    tab, ptab = _precompute(embedding, wp, bp, pe)
    tab3 = tab.reshape(V, 8, 128)
    ptab3 = ptab.reshape(L, 8, 128)
    ids1 = input_ids.astype(jnp.int32).reshape(B * L // CH, CH)
    return _make_sc_lookup()(ids1, tab3, ptab3)
